# baseline (speedup 1.0000x reference)
import functools

import jax
import jax.numpy as jnp
import numpy as np
from jax import lax
from jax.experimental import pallas as pl
from jax.experimental.pallas import tpu as pltpu
from jax.experimental.pallas import tpu_sc as plsc

M = 2048
DP = 128
R = 5000
NP = 5120
NT = 1024
BIGIDX = 2147483647


def _tf_rounds(x0, x1, rots):
    for r in rots:
        x0 = x0 + x1
        x1 = (x1 << np.uint32(r)) | (x1 >> np.uint32(32 - r))
        x1 = x0 ^ x1
    return x0, x1


def _gumbel_tile(flat_u32):
    k1 = np.uint32(0)
    k2 = np.uint32(42)
    k3 = k1 ^ k2 ^ np.uint32(0x1BD11BDA)
    rot0 = (13, 15, 26, 6)
    rot1 = (17, 29, 16, 24)
    x0 = jnp.zeros_like(flat_u32) + k1
    x1 = flat_u32 + k2
    x0, x1 = _tf_rounds(x0, x1, rot0)
    x0 = x0 + k2
    x1 = x1 + k3 + np.uint32(1)
    x0, x1 = _tf_rounds(x0, x1, rot1)
    x0 = x0 + k3
    x1 = x1 + k1 + np.uint32(2)
    x0, x1 = _tf_rounds(x0, x1, rot0)
    x0 = x0 + k1
    x1 = x1 + k2 + np.uint32(3)
    x0, x1 = _tf_rounds(x0, x1, rot1)
    x0 = x0 + k2
    x1 = x1 + k3 + np.uint32(4)
    x0, x1 = _tf_rounds(x0, x1, rot0)
    x0 = x0 + k3
    x1 = x1 + k1 + np.uint32(5)
    bits = x0 ^ x1
    float_bits = (bits >> np.uint32(9)) | np.uint32(0x3F800000)
    f = lax.bitcast_convert_type(float_bits, jnp.float32) - np.float32(1.0)
    tiny = np.float32(np.finfo(np.float32).tiny)
    u = jnp.maximum(tiny, f * (np.float32(1.0) - tiny) + tiny)
    return -jnp.log(-jnp.log(u))


def _stage1_body(x_ref, nz_ref, sv_ref, s2_ref, ts_ref,
                 bv_ref, bi_ref, nt_ref, x2_ref):
    j = pl.program_id(0)

    @pl.when(j == 0)
    def _():
        x2_ref[...] = jnp.sum(x_ref[...] ** 2, axis=1, keepdims=True)

    ab = lax.dot_general(x_ref[...], sv_ref[...],
                         (((1,), (1,)), ((), ())),
                         preferred_element_type=jnp.float32)
    sq = (x2_ref[...] + s2_ref[0:1, :]) - 2.0 * ab
    dist = jnp.maximum(sq, 0.0)
    mask = dist < ts_ref[:, 0:1]
    row = lax.broadcasted_iota(jnp.int32, (M, NT), 0)
    colg = lax.broadcasted_iota(jnp.int32, (M, NT), 1) + j * NT
    flat = (row * (R + 1) + colg).astype(jnp.uint32)
    g = _gumbel_tile(flat)
    cand = jnp.where(mask & (colg < R), g, -jnp.inf)
    tile_max = jnp.max(cand, axis=1, keepdims=True)
    tile_arg = jnp.min(jnp.where(cand == tile_max, colg, BIGIDX),
                       axis=1, keepdims=True)

    @pl.when(j == 0)
    def _():
        bv_ref[...] = tile_max
        bi_ref[...] = tile_arg
        nt_ref[...] = ts_ref[:, 1:2] * nz_ref[...]

    @pl.when(j > 0)
    def _():
        better = tile_max > bv_ref[...]
        bv_ref[...] = jnp.maximum(bv_ref[...], tile_max)
        bi_ref[...] = jnp.where(better, tile_arg, bi_ref[...])


SVOFF = 2 * M // NT


def _stage1(table, s2rep, tscol):
    return pl.pallas_call(
        _stage1_body,
        grid=(NP // NT,),
        in_specs=[
            pl.BlockSpec((M, DP), lambda j: (0, 0)),
            pl.BlockSpec((M, DP), lambda j: (1, 0)),
            pl.BlockSpec((NT, DP), lambda j: (SVOFF + j, 0)),
            pl.BlockSpec((8, NT), lambda j: (0, j)),
            pl.BlockSpec((M, 2), lambda j: (0, 0)),
        ],
        scratch_shapes=[pltpu.VMEM((M, 1), jnp.float32)],
        out_specs=[
            pl.BlockSpec((M, 1), lambda j: (0, 0)),
            pl.BlockSpec((M, 1), lambda j: (0, 0)),
            pl.BlockSpec((M, DP), lambda j: (0, 0)),
        ],
        out_shape=[
            jax.ShapeDtypeStruct((M, 1), jnp.float32),
            jax.ShapeDtypeStruct((M, 1), jnp.int32),
            jax.ShapeDtypeStruct((M, DP), jnp.float32),
        ],
        compiler_params=pltpu.CompilerParams(
            dimension_semantics=("arbitrary",)),
    )(table, table, table, s2rep, tscol)


def _sc_body(table, bestval, bestidx, noiset, out, bv_v, bi_v, idx_v,
             rows_v, nt_v, sem):
    c = lax.axis_index("c")
    s = lax.axis_index("s")
    wid = s * 2 + c
    rows = M // 32
    base = wid * rows
    pltpu.sync_copy(bestval.at[pl.ds(base, rows)], bv_v)
    pltpu.sync_copy(bestidx.at[pl.ds(base, rows)], bi_v)
    for ch in range(rows // 16):
        sl = pl.ds(ch * 16, 16)
        has_pick = bv_v[sl] > -jnp.inf
        rowid = lax.iota(jnp.int32, 16) + (base + ch * 16)
        idx_v[sl] = jnp.where(has_pick, bi_v[sl] + 2 * M, rowid)
    pltpu.async_copy(table.at[idx_v], rows_v, sem).wait()
    pltpu.sync_copy(noiset.at[pl.ds(base, rows)], nt_v)

    def row_body(r, carry):
        for vv in range(DP // 16):
            sl = pl.ds(vv * 16, 16)
            rows_v[r, sl] = rows_v[r, sl] + nt_v[r, sl]
        return carry

    lax.fori_loop(0, rows, row_body, 0)
    pltpu.sync_copy(rows_v, out.at[pl.ds(base, rows)])


def _sc_gather(table, bestval_flat, bestidx_flat, noise_t):
    rows = M // 32
    mesh = plsc.VectorSubcoreMesh(core_axis_name="c", subcore_axis_name="s")
    fn = functools.partial(
        pl.kernel,
        out_type=jax.ShapeDtypeStruct((M, DP), jnp.float32),
        mesh=mesh,
        scratch_types=[
            pltpu.VMEM((rows,), jnp.float32),
            pltpu.VMEM((rows,), jnp.int32),
            pltpu.VMEM((rows,), jnp.int32),
            pltpu.VMEM((rows, DP), jnp.float32),
            pltpu.VMEM((rows, DP), jnp.float32),
            pltpu.SemaphoreType.DMA,
        ],
    )(_sc_body)
    return fn(table, bestval_flat, bestidx_flat, noise_t)


def kernel(x_start, t, noise, sampled_values, distance_schedule, noise_schedule):
    b, s, d = x_start.shape
    r = sampled_values.shape[0]
    x_flat = x_start.reshape(b * s, d)
    s2 = jnp.sum(sampled_values ** 2, axis=1)
    thr = distance_schedule[t]
    thr2_row = jnp.repeat(thr ** 2, s)
    scale_row = jnp.repeat(noise_schedule[t], s)
    tscol = jnp.concatenate([thr2_row[:, None], scale_row[:, None]], axis=1)

    table = jnp.pad(
        jnp.concatenate([x_flat, noise.reshape(b * s, d), sampled_values],
                        axis=0),
        ((0, NP - r), (0, DP - d)))
    s2rep = jnp.broadcast_to(jnp.pad(s2, (0, NP - r))[None, :], (8, NP))

    bestval, bestidx, noise_t = _stage1(table, s2rep, tscol)

    out_pad = _sc_gather(table, bestval.reshape(b * s),
                         bestidx.reshape(b * s), noise_t)
    return out_pad[:, :d].reshape(b, s, d)


# speedup vs baseline: 1.2687x; 1.2687x over previous
import functools

import jax
import jax.numpy as jnp
import numpy as np
from jax import lax
from jax.experimental import pallas as pl
from jax.experimental.pallas import tpu as pltpu
from jax.experimental.pallas import tpu_sc as plsc

M = 2048
DP = 128
R = 5000
NP = 5120
NT = 512
BIGIDX = 2147483647


def _tf_rounds(x0, x1, rots):
    for r in rots:
        x0 = x0 + x1
        x1 = (x1 << np.uint32(r)) | (x1 >> np.uint32(32 - r))
        x1 = x0 ^ x1
    return x0, x1


def _gumbel_tile(flat_u32):
    k1 = np.uint32(0)
    k2 = np.uint32(42)
    k3 = k1 ^ k2 ^ np.uint32(0x1BD11BDA)
    rot0 = (13, 15, 26, 6)
    rot1 = (17, 29, 16, 24)
    x0 = jnp.zeros_like(flat_u32) + k1
    x1 = flat_u32 + k2
    x0, x1 = _tf_rounds(x0, x1, rot0)
    x0 = x0 + k2
    x1 = x1 + k3 + np.uint32(1)
    x0, x1 = _tf_rounds(x0, x1, rot1)
    x0 = x0 + k3
    x1 = x1 + k1 + np.uint32(2)
    x0, x1 = _tf_rounds(x0, x1, rot0)
    x0 = x0 + k1
    x1 = x1 + k2 + np.uint32(3)
    x0, x1 = _tf_rounds(x0, x1, rot1)
    x0 = x0 + k2
    x1 = x1 + k3 + np.uint32(4)
    x0, x1 = _tf_rounds(x0, x1, rot0)
    x0 = x0 + k3
    x1 = x1 + k1 + np.uint32(5)
    bits = x0 ^ x1
    float_bits = (bits >> np.uint32(9)) | np.uint32(0x3F800000)
    f = lax.bitcast_convert_type(float_bits, jnp.float32) - np.float32(1.0)
    tiny = np.float32(np.finfo(np.float32).tiny)
    u = jnp.maximum(tiny, f * (np.float32(1.0) - tiny) + tiny)
    return -jnp.log(-jnp.log(u))


def _stage1_body(x_ref, nz_ref, sv_ref, s2_ref, ts_ref,
                 bv_ref, bi_ref, nt_ref, x2_ref):
    j = pl.program_id(0)

    @pl.when(j == 0)
    def _():
        x2_ref[...] = jnp.sum(x_ref[...] ** 2, axis=1, keepdims=True)

    ab = lax.dot_general(x_ref[...], sv_ref[...],
                         (((1,), (1,)), ((), ())),
                         preferred_element_type=jnp.float32)
    sq = (x2_ref[...] + s2_ref[0:1, :]) - 2.0 * ab
    dist = jnp.maximum(sq, 0.0)
    mask = dist < ts_ref[:, 0:1]
    row = lax.broadcasted_iota(jnp.int32, (M, NT), 0)
    colg = lax.broadcasted_iota(jnp.int32, (M, NT), 1) + j * NT
    flat = (row * (R + 1) + colg).astype(jnp.uint32)
    g = _gumbel_tile(flat)
    cand = jnp.where(mask & (colg < R), g, -jnp.inf)
    tile_max = jnp.max(cand, axis=1, keepdims=True)
    tile_arg = jnp.min(jnp.where(cand == tile_max, colg, BIGIDX),
                       axis=1, keepdims=True)

    @pl.when(j == 0)
    def _():
        bv_ref[...] = tile_max
        bi_ref[...] = tile_arg
        nt_ref[...] = ts_ref[:, 1:2] * nz_ref[...]

    @pl.when(j > 0)
    def _():
        better = tile_max > bv_ref[...]
        bv_ref[...] = jnp.maximum(bv_ref[...], tile_max)
        bi_ref[...] = jnp.where(better, tile_arg, bi_ref[...])


SVOFF = 2 * M // NT


def _stage1(table, s2rep, tscol):
    return pl.pallas_call(
        _stage1_body,
        grid=(NP // NT,),
        in_specs=[
            pl.BlockSpec((M, DP), lambda j: (0, 0)),
            pl.BlockSpec((M, DP), lambda j: (1, 0)),
            pl.BlockSpec((NT, DP), lambda j: (SVOFF + j, 0)),
            pl.BlockSpec((1, NT), lambda j: (0, j)),
            pl.BlockSpec((M, 2), lambda j: (0, 0)),
        ],
        scratch_shapes=[pltpu.VMEM((M, 1), jnp.float32)],
        out_specs=[
            pl.BlockSpec((M, 1), lambda j: (0, 0)),
            pl.BlockSpec((M, 1), lambda j: (0, 0)),
            pl.BlockSpec((M, DP), lambda j: (0, 0)),
        ],
        out_shape=[
            jax.ShapeDtypeStruct((M, 1), jnp.float32),
            jax.ShapeDtypeStruct((M, 1), jnp.int32),
            jax.ShapeDtypeStruct((M, DP), jnp.float32),
        ],
        compiler_params=pltpu.CompilerParams(
            dimension_semantics=("arbitrary",)),
    )(table, table, table, s2rep, tscol)


def _sc_body(table, bestval, bestidx, noiset, out, bv_v, bi_v, idx_v,
             rows_v, nt_v, sem):
    c = lax.axis_index("c")
    s = lax.axis_index("s")
    wid = s * 2 + c
    rows = M // 32
    base = wid * rows
    pltpu.sync_copy(bestval.at[pl.ds(base, rows)], bv_v)
    pltpu.sync_copy(bestidx.at[pl.ds(base, rows)], bi_v)
    for ch in range(rows // 16):
        sl = pl.ds(ch * 16, 16)
        has_pick = bv_v[sl] > -jnp.inf
        rowid = lax.iota(jnp.int32, 16) + (base + ch * 16)
        idx_v[sl] = jnp.where(has_pick, bi_v[sl] + 2 * M, rowid)
    gather = pltpu.async_copy(table.at[idx_v], rows_v, sem)
    pltpu.sync_copy(noiset.at[pl.ds(base, rows)], nt_v)
    gather.wait()

    def row_body(r, carry):
        for vv in range(DP // 16):
            sl = pl.ds(vv * 16, 16)
            rows_v[r, sl] = rows_v[r, sl] + nt_v[r, sl]
        return carry

    lax.fori_loop(0, rows, row_body, 0)
    pltpu.sync_copy(rows_v, out.at[pl.ds(base, rows)])


def _sc_gather(table, bestval_flat, bestidx_flat, noise_t):
    rows = M // 32
    mesh = plsc.VectorSubcoreMesh(core_axis_name="c", subcore_axis_name="s")
    fn = functools.partial(
        pl.kernel,
        out_type=jax.ShapeDtypeStruct((M, DP), jnp.float32),
        mesh=mesh,
        scratch_types=[
            pltpu.VMEM((rows,), jnp.float32),
            pltpu.VMEM((rows,), jnp.int32),
            pltpu.VMEM((rows,), jnp.int32),
            pltpu.VMEM((rows, DP), jnp.float32),
            pltpu.VMEM((rows, DP), jnp.float32),
            pltpu.SemaphoreType.DMA,
        ],
    )(_sc_body)
    return fn(table, bestval_flat, bestidx_flat, noise_t)


def kernel(x_start, t, noise, sampled_values, distance_schedule, noise_schedule):
    b, s, d = x_start.shape
    r = sampled_values.shape[0]
    x_flat = x_start.reshape(b * s, d)
    s2 = jnp.sum(sampled_values ** 2, axis=1)
    thr = distance_schedule[t]
    thr2_row = jnp.repeat(thr ** 2, s)
    scale_row = jnp.repeat(noise_schedule[t], s)
    tscol = jnp.concatenate([thr2_row[:, None], scale_row[:, None]], axis=1)

    table = jnp.pad(
        jnp.concatenate([x_flat, noise.reshape(b * s, d), sampled_values],
                        axis=0),
        ((0, NP - r), (0, DP - d)))
    s2rep = jnp.pad(s2, (0, NP - r))[None, :]

    bestval, bestidx, noise_t = _stage1(table, s2rep, tscol)

    out_pad = _sc_gather(table, bestval.reshape(b * s),
                         bestidx.reshape(b * s), noise_t)
    return out_pad[:, :d].reshape(b, s, d)


# speedup vs baseline: 1.2688x; 1.0001x over previous
import functools

import jax
import jax.numpy as jnp
import numpy as np
from jax import lax
from jax.experimental import pallas as pl
from jax.experimental.pallas import tpu as pltpu
from jax.experimental.pallas import tpu_sc as plsc

M = 2048
DP = 128
R = 5000
NP = 5120
NT = 512
BIGIDX = 2147483647


def _tf_rounds(x0, x1, rots):
    for r in rots:
        x0 = x0 + x1
        x1 = (x1 << np.uint32(r)) | (x1 >> np.uint32(32 - r))
        x1 = x0 ^ x1
    return x0, x1


def _gumbel_tile(flat_u32):
    k1 = np.uint32(0)
    k2 = np.uint32(42)
    k3 = k1 ^ k2 ^ np.uint32(0x1BD11BDA)
    rot0 = (13, 15, 26, 6)
    rot1 = (17, 29, 16, 24)
    x0 = jnp.zeros_like(flat_u32) + k1
    x1 = flat_u32 + k2
    x0, x1 = _tf_rounds(x0, x1, rot0)
    x0 = x0 + k2
    x1 = x1 + k3 + np.uint32(1)
    x0, x1 = _tf_rounds(x0, x1, rot1)
    x0 = x0 + k3
    x1 = x1 + k1 + np.uint32(2)
    x0, x1 = _tf_rounds(x0, x1, rot0)
    x0 = x0 + k1
    x1 = x1 + k2 + np.uint32(3)
    x0, x1 = _tf_rounds(x0, x1, rot1)
    x0 = x0 + k2
    x1 = x1 + k3 + np.uint32(4)
    x0, x1 = _tf_rounds(x0, x1, rot0)
    x0 = x0 + k3
    x1 = x1 + k1 + np.uint32(5)
    bits = x0 ^ x1
    float_bits = (bits >> np.uint32(9)) | np.uint32(0x3F800000)
    f = lax.bitcast_convert_type(float_bits, jnp.float32) - np.float32(1.0)
    tiny = np.float32(np.finfo(np.float32).tiny)
    u = jnp.maximum(tiny, f * (np.float32(1.0) - tiny) + tiny)
    return -jnp.log(-jnp.log(u))


def _stage1_body(x_ref, nz_ref, sv_ref, s2_ref, ts_ref,
                 bv_ref, bi_ref, nt_ref, x2_ref):
    j = pl.program_id(0)

    @pl.when(j == 0)
    def _():
        x2_ref[...] = jnp.sum(x_ref[...] ** 2, axis=1, keepdims=True)

    ab = lax.dot_general(x_ref[...], sv_ref[...],
                         (((1,), (1,)), ((), ())),
                         preferred_element_type=jnp.float32)
    sq = (x2_ref[...] + s2_ref[0:1, :]) - 2.0 * ab
    dist = jnp.maximum(sq, 0.0)
    mask = dist < ts_ref[:, 0:1]
    row = lax.broadcasted_iota(jnp.int32, (M, NT), 0)
    colg = lax.broadcasted_iota(jnp.int32, (M, NT), 1) + j * NT
    flat = (row * (R + 1) + colg).astype(jnp.uint32)
    g = _gumbel_tile(flat)
    cand = jnp.where(mask, g, -jnp.inf)
    tile_max = jnp.max(cand, axis=1, keepdims=True)
    tile_arg = jnp.min(jnp.where(cand == tile_max, colg, BIGIDX),
                       axis=1, keepdims=True)

    @pl.when(j == 0)
    def _():
        bv_ref[...] = tile_max
        bi_ref[...] = tile_arg
        nt_ref[...] = ts_ref[:, 1:2] * nz_ref[...]

    @pl.when(j > 0)
    def _():
        better = tile_max > bv_ref[...]
        bv_ref[...] = jnp.maximum(bv_ref[...], tile_max)
        bi_ref[...] = jnp.where(better, tile_arg, bi_ref[...])


SVOFF = 2 * M // NT


def _stage1(table, s2rep, tscol):
    return pl.pallas_call(
        _stage1_body,
        grid=(NP // NT,),
        in_specs=[
            pl.BlockSpec((M, DP), lambda j: (0, 0)),
            pl.BlockSpec((M, DP), lambda j: (1, 0)),
            pl.BlockSpec((NT, DP), lambda j: (SVOFF + j, 0)),
            pl.BlockSpec((1, NT), lambda j: (0, j)),
            pl.BlockSpec((M, 2), lambda j: (0, 0)),
        ],
        scratch_shapes=[pltpu.VMEM((M, 1), jnp.float32)],
        out_specs=[
            pl.BlockSpec((M, 1), lambda j: (0, 0)),
            pl.BlockSpec((M, 1), lambda j: (0, 0)),
            pl.BlockSpec((M, DP), lambda j: (0, 0)),
        ],
        out_shape=[
            jax.ShapeDtypeStruct((M, 1), jnp.float32),
            jax.ShapeDtypeStruct((M, 1), jnp.int32),
            jax.ShapeDtypeStruct((M, DP), jnp.float32),
        ],
        compiler_params=pltpu.CompilerParams(
            dimension_semantics=("arbitrary",)),
    )(table, table, table, s2rep, tscol)


def _sc_body(table, bestval, bestidx, noiset, out, bv_v, bi_v, idx_v,
             rows_v, nt_v, sem):
    c = lax.axis_index("c")
    s = lax.axis_index("s")
    wid = s * 2 + c
    rows = M // 32
    base = wid * rows
    pltpu.sync_copy(bestval.at[pl.ds(base, rows)], bv_v)
    pltpu.sync_copy(bestidx.at[pl.ds(base, rows)], bi_v)
    for ch in range(rows // 16):
        sl = pl.ds(ch * 16, 16)
        has_pick = bv_v[sl] > -jnp.inf
        rowid = lax.iota(jnp.int32, 16) + (base + ch * 16)
        idx_v[sl] = jnp.where(has_pick, bi_v[sl] + 2 * M, rowid)
    gather = pltpu.async_copy(table.at[idx_v], rows_v, sem)
    pltpu.sync_copy(noiset.at[pl.ds(base, rows)], nt_v)
    gather.wait()

    def row_body(r, carry):
        for vv in range(DP // 16):
            sl = pl.ds(vv * 16, 16)
            rows_v[r, sl] = rows_v[r, sl] + nt_v[r, sl]
        return carry

    lax.fori_loop(0, rows, row_body, 0)
    pltpu.sync_copy(rows_v, out.at[pl.ds(base, rows)])


def _sc_gather(table, bestval_flat, bestidx_flat, noise_t):
    rows = M // 32
    mesh = plsc.VectorSubcoreMesh(core_axis_name="c", subcore_axis_name="s")
    fn = functools.partial(
        pl.kernel,
        out_type=jax.ShapeDtypeStruct((M, DP), jnp.float32),
        mesh=mesh,
        scratch_types=[
            pltpu.VMEM((rows,), jnp.float32),
            pltpu.VMEM((rows,), jnp.int32),
            pltpu.VMEM((rows,), jnp.int32),
            pltpu.VMEM((rows, DP), jnp.float32),
            pltpu.VMEM((rows, DP), jnp.float32),
            pltpu.SemaphoreType.DMA,
        ],
    )(_sc_body)
    return fn(table, bestval_flat, bestidx_flat, noise_t)


def kernel(x_start, t, noise, sampled_values, distance_schedule, noise_schedule):
    b, s, d = x_start.shape
    r = sampled_values.shape[0]
    x_flat = x_start.reshape(b * s, d)
    s2 = jnp.sum(sampled_values ** 2, axis=1)
    thr = distance_schedule[t]
    thr2_row = jnp.repeat(thr ** 2, s)
    scale_row = jnp.repeat(noise_schedule[t], s)
    tscol = jnp.concatenate([thr2_row[:, None], scale_row[:, None]], axis=1)

    table = jnp.pad(
        jnp.concatenate([x_flat, noise.reshape(b * s, d), sampled_values],
                        axis=0),
        ((0, NP - r), (0, DP - d)))
    s2rep = jnp.pad(s2, (0, NP - r), constant_values=jnp.inf)[None, :]

    bestval, bestidx, noise_t = _stage1(table, s2rep, tscol)

    out_pad = _sc_gather(table, bestval.reshape(b * s),
                         bestidx.reshape(b * s), noise_t)
    return out_pad[:, :d].reshape(b, s, d)
